# baseline (device time: 46866 ns/iter reference)
import jax
import jax.numpy as jnp
from jax import lax
from jax.experimental import pallas as pl
from jax.experimental.pallas import tpu as pltpu

N_DEV = 8
B, SQ_PER, D = 2, 128, 512
HQ_PER, DH = 8, 64
BH = B * HQ_PER

X_SCALE = 4.5 / 127.0


def kernel(x, Wq, Wo, K_ext, V_ext):
    def body(x_ref, wq_ref, wo_ref, k_ref, v_ref, out_ref,
             x_i8_ref, ag_ref, rs_send_ref, rs_recv_ref,
             ag_send_sems, ag_recv_sems, rs_send_sems, rs_recv_sems):
        my = lax.axis_index("i")

        barrier_sem = pltpu.get_barrier_semaphore()
        for d in range(1, N_DEV):
            pl.semaphore_signal(
                barrier_sem, inc=1,
                device_id=((my + d) % N_DEV,),
                device_id_type=pl.DeviceIdType.MESH,
            )
        pl.semaphore_wait(barrier_sem, N_DEV - 1)

        x_i8_ref[...] = jnp.clip(
            jnp.rint(x_ref[...] * (1.0 / X_SCALE)), -127, 127
        ).astype(jnp.int8)
        ag_sends = []
        for d in range(1, N_DEV):
            rdma = pltpu.make_async_remote_copy(
                src_ref=x_i8_ref,
                dst_ref=ag_ref.at[d - 1],
                send_sem=ag_send_sems.at[d - 1],
                recv_sem=ag_recv_sems.at[d - 1],
                device_id=((my + d) % N_DEV,),
                device_id_type=pl.DeviceIdType.MESH,
            )
            rdma.start()
            ag_sends.append(rdma)

        k_loc = k_ref[:, :, pl.ds(HQ_PER * my, HQ_PER), :]
        v_loc = v_ref[:, :, pl.ds(HQ_PER * my, HQ_PER), :]
        k_loc = k_loc.transpose(0, 2, 1, 3).reshape(BH, 128, DH)
        v_loc = v_loc.transpose(0, 2, 1, 3).reshape(BH, 128, DH)
        k_loc = k_loc.astype(jnp.bfloat16)
        v_loc = v_loc.astype(jnp.bfloat16)

        wq = wq_ref[...].astype(jnp.bfloat16)
        wo = wo_ref[...].astype(jnp.bfloat16)

        def contribution(xc):
            q = jnp.dot(xc.reshape(B * SQ_PER, D), wq,
                        preferred_element_type=jnp.float32)
            q = (q * 0.125).astype(jnp.bfloat16)
            q = q.reshape(B, SQ_PER, HQ_PER, DH).transpose(0, 2, 1, 3)
            q = q.reshape(BH, SQ_PER, DH)
            s = lax.dot_general(
                q, k_loc, (((2,), (2,)), ((0,), (0,))),
                preferred_element_type=jnp.float32)
            p = jnp.exp(s).astype(jnp.bfloat16)
            l = jnp.sum(p, axis=-1, keepdims=True, dtype=jnp.float32)
            y = lax.dot_general(
                p, v_loc, (((2,), (1,)), ((0,), (0,))),
                preferred_element_type=jnp.float32)
            y = (y * (1.0 / l)).astype(jnp.bfloat16).reshape(
                B, HQ_PER, SQ_PER, DH)
            y = y.transpose(0, 2, 1, 3).reshape(B * SQ_PER, D)
            return jnp.dot(y, wo, preferred_element_type=jnp.float32)

        x_scale_bf = jnp.full((1, 1, 1), X_SCALE, jnp.bfloat16)

        def dequant_x(slot_val):
            return slot_val.astype(jnp.bfloat16) * x_scale_bf

        acc = contribution(dequant_x(x_i8_ref[...]))

        rs_sends = []
        for s in range(N_DEV - 1):
            recv = pltpu.make_async_remote_copy(
                src_ref=x_i8_ref,
                dst_ref=ag_ref.at[s],
                send_sem=ag_send_sems.at[s],
                recv_sem=ag_recv_sems.at[s],
                device_id=(my,),
                device_id_type=pl.DeviceIdType.MESH,
            )
            recv.wait_recv()
            rs_send_ref[s] = contribution(dequant_x(ag_ref[s])).astype(
                jnp.bfloat16).reshape(B, SQ_PER, D)
            rdma = pltpu.make_async_remote_copy(
                src_ref=rs_send_ref.at[s],
                dst_ref=rs_recv_ref.at[s],
                send_sem=rs_send_sems.at[s],
                recv_sem=rs_recv_sems.at[s],
                device_id=((my - 1 - s) % N_DEV,),
                device_id_type=pl.DeviceIdType.MESH,
            )
            rdma.start()
            rs_sends.append(rdma)

        for q_ in range(N_DEV - 1):
            recv = pltpu.make_async_remote_copy(
                src_ref=rs_send_ref.at[q_],
                dst_ref=rs_recv_ref.at[q_],
                send_sem=rs_send_sems.at[q_],
                recv_sem=rs_recv_sems.at[q_],
                device_id=(my,), device_id_type=pl.DeviceIdType.MESH,
            )
            recv.wait_recv()
            acc = acc + rs_recv_ref[q_].reshape(
                B * SQ_PER, D).astype(jnp.float32)
        out_ref[...] = acc.reshape(B, SQ_PER, D)

        for rdma in ag_sends + rs_sends:
            rdma.wait_send()

    return pl.pallas_call(
        body,
        out_shape=jax.ShapeDtypeStruct((B, SQ_PER, D), jnp.float32),
        in_specs=[pl.BlockSpec(memory_space=pltpu.VMEM)] * 5,
        out_specs=pl.BlockSpec(memory_space=pltpu.VMEM),
        scratch_shapes=[
            pltpu.VMEM((B, SQ_PER, D), jnp.int8),
            pltpu.VMEM((N_DEV - 1, B, SQ_PER, D), jnp.int8),
            pltpu.VMEM((N_DEV - 1, B, SQ_PER, D), jnp.bfloat16),
            pltpu.VMEM((N_DEV - 1, B, SQ_PER, D), jnp.bfloat16),
            pltpu.SemaphoreType.DMA((N_DEV - 1,)),
            pltpu.SemaphoreType.DMA((N_DEV - 1,)),
            pltpu.SemaphoreType.DMA((N_DEV - 1,)),
            pltpu.SemaphoreType.DMA((N_DEV - 1,)),
        ],
        compiler_params=pltpu.CompilerParams(collective_id=0),
    )(x, Wq, Wo, K_ext, V_ext)


# device time: 46321 ns/iter; 1.0118x vs baseline; 1.0118x over previous
import jax
import jax.numpy as jnp
from jax import lax
from jax.experimental import pallas as pl
from jax.experimental.pallas import tpu as pltpu

N_DEV = 8
B, SQ_PER, D = 2, 128, 512
HQ_PER, DH = 8, 64
BH = B * HQ_PER

X_SCALE = 4.5 / 127.0


def kernel(x, Wq, Wo, K_ext, V_ext):
    def body(x_ref, wq_ref, wo_ref, k_ref, v_ref, out_ref,
             x_i8_ref, ag_ref, rs_send_ref, rs_recv_ref,
             sc_send_ref, sc_recv_ref,
             ag_send_sems, ag_recv_sems, rs_send_sems, rs_recv_sems,
             sc_send_sems, sc_recv_sems):
        my = lax.axis_index("i")

        barrier_sem = pltpu.get_barrier_semaphore()
        for d in range(1, N_DEV):
            pl.semaphore_signal(
                barrier_sem, inc=1,
                device_id=((my + d) % N_DEV,),
                device_id_type=pl.DeviceIdType.MESH,
            )
        pl.semaphore_wait(barrier_sem, N_DEV - 1)

        x_i8_ref[...] = jnp.clip(
            jnp.rint(x_ref[...] * (1.0 / X_SCALE)), -127, 127
        ).astype(jnp.int8)
        ag_sends = []
        for d in range(1, N_DEV):
            rdma = pltpu.make_async_remote_copy(
                src_ref=x_i8_ref,
                dst_ref=ag_ref.at[d - 1],
                send_sem=ag_send_sems.at[d - 1],
                recv_sem=ag_recv_sems.at[d - 1],
                device_id=((my + d) % N_DEV,),
                device_id_type=pl.DeviceIdType.MESH,
            )
            rdma.start()
            ag_sends.append(rdma)

        k_loc = k_ref[:, :, pl.ds(HQ_PER * my, HQ_PER), :]
        v_loc = v_ref[:, :, pl.ds(HQ_PER * my, HQ_PER), :]
        k_loc = k_loc.transpose(0, 2, 1, 3).reshape(BH, 128, DH)
        v_loc = v_loc.transpose(0, 2, 1, 3).reshape(BH, 128, DH)
        k_loc = k_loc.astype(jnp.bfloat16)
        v_loc = v_loc.astype(jnp.bfloat16)

        wq = wq_ref[...].astype(jnp.bfloat16)
        wo = wo_ref[...].astype(jnp.bfloat16)

        def contribution(xc):
            q = jnp.dot(xc.reshape(B * SQ_PER, D), wq,
                        preferred_element_type=jnp.float32)
            q = (q * 0.125).astype(jnp.bfloat16)
            q = q.reshape(B, SQ_PER, HQ_PER, DH).transpose(0, 2, 1, 3)
            q = q.reshape(BH, SQ_PER, DH)
            s = lax.dot_general(
                q, k_loc, (((2,), (2,)), ((0,), (0,))),
                preferred_element_type=jnp.float32)
            p = jnp.exp(s).astype(jnp.bfloat16)
            l = jnp.sum(p, axis=-1, keepdims=True, dtype=jnp.float32)
            y = lax.dot_general(
                p, v_loc, (((2,), (1,)), ((0,), (0,))),
                preferred_element_type=jnp.float32)
            y = (y * (1.0 / l)).astype(jnp.bfloat16).reshape(
                B, HQ_PER, SQ_PER, DH)
            y = y.transpose(0, 2, 1, 3).reshape(B * SQ_PER, D)
            return jnp.dot(y, wo, preferred_element_type=jnp.float32)

        x_scale_bf = jnp.full((1, 1, 1), X_SCALE, jnp.bfloat16)

        def dequant_x(slot_val):
            return slot_val.astype(jnp.bfloat16) * x_scale_bf

        def collect(acc, q_):
            for dst, rsem, src, ssem in (
                (rs_recv_ref.at[q_], rs_recv_sems.at[q_],
                 rs_send_ref.at[q_], rs_send_sems.at[q_]),
                (sc_recv_ref.at[q_], sc_recv_sems.at[q_],
                 sc_send_ref.at[q_], sc_send_sems.at[q_]),
            ):
                recv = pltpu.make_async_remote_copy(
                    src_ref=src, dst_ref=dst, send_sem=ssem, recv_sem=rsem,
                    device_id=(my,), device_id_type=pl.DeviceIdType.MESH,
                )
                recv.wait_recv()
            scale = sc_recv_ref[q_][0, 0]
            return acc + rs_recv_ref[q_].reshape(
                B * SQ_PER, D).astype(jnp.float32) * scale

        acc = contribution(dequant_x(x_i8_ref[...]))

        rs_sends = []
        for s in range(N_DEV - 1):
            recv = pltpu.make_async_remote_copy(
                src_ref=x_i8_ref,
                dst_ref=ag_ref.at[s],
                send_sem=ag_send_sems.at[s],
                recv_sem=ag_recv_sems.at[s],
                device_id=(my,),
                device_id_type=pl.DeviceIdType.MESH,
            )
            recv.wait_recv()
            part = contribution(dequant_x(ag_ref[s])).astype(jnp.bfloat16)
            m1 = jnp.max(jnp.abs(part), axis=0).astype(jnp.float32)
            mx_arr = jnp.maximum(
                jnp.max(m1, keepdims=True), 1e-20).reshape(1, 1)
            inv = (127.0 / mx_arr).astype(jnp.bfloat16)
            rs_send_ref[s] = jnp.clip(
                jnp.rint(part * inv), -127, 127
            ).astype(jnp.int8).reshape(B, SQ_PER, D)
            sc_send_ref[s] = jnp.full((8, 128), mx_arr[0, 0] / 127.0,
                                      jnp.float32)
            dest = ((my - 1 - s) % N_DEV,)
            for src, dst, ssem, rsem in (
                (rs_send_ref.at[s], rs_recv_ref.at[s],
                 rs_send_sems.at[s], rs_recv_sems.at[s]),
                (sc_send_ref.at[s], sc_recv_ref.at[s],
                 sc_send_sems.at[s], sc_recv_sems.at[s]),
            ):
                rdma = pltpu.make_async_remote_copy(
                    src_ref=src, dst_ref=dst, send_sem=ssem, recv_sem=rsem,
                    device_id=dest, device_id_type=pl.DeviceIdType.MESH,
                )
                rdma.start()
                rs_sends.append(rdma)
            if s >= 2:
                acc = collect(acc, s - 2)

        for q_ in range(N_DEV - 3, N_DEV - 1):
            acc = collect(acc, q_)
        out_ref[...] = acc.reshape(B, SQ_PER, D)

        for rdma in ag_sends + rs_sends:
            rdma.wait_send()

    return pl.pallas_call(
        body,
        out_shape=jax.ShapeDtypeStruct((B, SQ_PER, D), jnp.float32),
        in_specs=[pl.BlockSpec(memory_space=pltpu.VMEM)] * 5,
        out_specs=pl.BlockSpec(memory_space=pltpu.VMEM),
        scratch_shapes=[
            pltpu.VMEM((B, SQ_PER, D), jnp.int8),
            pltpu.VMEM((N_DEV - 1, B, SQ_PER, D), jnp.int8),
            pltpu.VMEM((N_DEV - 1, B, SQ_PER, D), jnp.int8),
            pltpu.VMEM((N_DEV - 1, B, SQ_PER, D), jnp.int8),
            pltpu.VMEM((N_DEV - 1, 8, 128), jnp.float32),
            pltpu.VMEM((N_DEV - 1, 8, 128), jnp.float32),
            pltpu.SemaphoreType.DMA((N_DEV - 1,)),
            pltpu.SemaphoreType.DMA((N_DEV - 1,)),
            pltpu.SemaphoreType.DMA((N_DEV - 1,)),
            pltpu.SemaphoreType.DMA((N_DEV - 1,)),
            pltpu.SemaphoreType.DMA((N_DEV - 1,)),
            pltpu.SemaphoreType.DMA((N_DEV - 1,)),
        ],
        compiler_params=pltpu.CompilerParams(collective_id=0),
    )(x, Wq, Wo, K_ext, V_ext)


# device time: 42190 ns/iter; 1.1108x vs baseline; 1.0979x over previous
import jax
import jax.numpy as jnp
from jax import lax
from jax.experimental import pallas as pl
from jax.experimental.pallas import tpu as pltpu

N_DEV = 8
B, SQ_PER, D = 2, 128, 512
HQ_PER, DH = 8, 64
BH = B * HQ_PER

X_SCALE = 4.5 / 127.0


def kernel(x, Wq, Wo, K_ext, V_ext):
    def body(x_ref, wq_ref, wo_ref, k_ref, v_ref, out_ref,
             x_i8_ref, ag_ref, rs_send_ref, rs_recv_ref,
             sc_send_ref, sc_recv_ref,
             ag_send_sems, ag_recv_sems, rs_send_sems, rs_recv_sems,
             sc_send_sems, sc_recv_sems):
        my = lax.axis_index("i")

        barrier_sem = pltpu.get_barrier_semaphore()
        for d in range(1, N_DEV):
            pl.semaphore_signal(
                barrier_sem, inc=1,
                device_id=((my + d) % N_DEV,),
                device_id_type=pl.DeviceIdType.MESH,
            )
        pl.semaphore_wait(barrier_sem, N_DEV - 1)

        x_i8_ref[...] = jnp.clip(
            jnp.rint(x_ref[...] * (1.0 / X_SCALE)), -127, 127
        ).astype(jnp.int8)
        ag_sends = []
        for d in range(1, N_DEV):
            rdma = pltpu.make_async_remote_copy(
                src_ref=x_i8_ref,
                dst_ref=ag_ref.at[d - 1],
                send_sem=ag_send_sems.at[d - 1],
                recv_sem=ag_recv_sems.at[d - 1],
                device_id=((my + d) % N_DEV,),
                device_id_type=pl.DeviceIdType.MESH,
            )
            rdma.start()
            ag_sends.append(rdma)

        k_loc = k_ref[:, :, pl.ds(HQ_PER * my, HQ_PER), :]
        v_loc = v_ref[:, :, pl.ds(HQ_PER * my, HQ_PER), :]
        k_loc = k_loc.transpose(0, 2, 1, 3).reshape(BH, 128, DH)
        v_loc = v_loc.transpose(0, 2, 1, 3).reshape(BH, 128, DH)
        k_loc = k_loc.astype(jnp.bfloat16)
        v_loc = v_loc.astype(jnp.bfloat16)

        wq = wq_ref[...].astype(jnp.bfloat16)
        wo = wo_ref[...].astype(jnp.bfloat16)

        def contribution(xc):
            q = jnp.dot(xc.reshape(B * SQ_PER, D), wq,
                        preferred_element_type=jnp.float32)
            q = (q * 0.125).astype(jnp.bfloat16)
            q = q.reshape(B, SQ_PER, HQ_PER, DH).transpose(0, 2, 1, 3)
            q = q.reshape(BH, SQ_PER, DH)
            s = lax.dot_general(
                q, k_loc, (((2,), (2,)), ((0,), (0,))),
                preferred_element_type=jnp.float32)
            p = jnp.exp(s).astype(jnp.bfloat16)
            l = jnp.sum(p, axis=-1, keepdims=True, dtype=jnp.float32)
            y = lax.dot_general(
                p, v_loc, (((2,), (1,)), ((0,), (0,))),
                preferred_element_type=jnp.float32)
            y = (y * (1.0 / l)).astype(jnp.bfloat16).reshape(
                B, HQ_PER, SQ_PER, DH)
            y = y.transpose(0, 2, 1, 3).reshape(B * SQ_PER, D)
            return jnp.dot(y, wo, preferred_element_type=jnp.float32)

        x_scale_bf = jnp.full((1, 1, 1), X_SCALE, jnp.bfloat16)

        def dequant_x(slot_val):
            return slot_val.astype(jnp.bfloat16) * x_scale_bf

        acc = contribution(dequant_x(x_i8_ref[...]))

        rs_sends = []
        for s in range(N_DEV - 1):
            recv = pltpu.make_async_remote_copy(
                src_ref=x_i8_ref,
                dst_ref=ag_ref.at[s],
                send_sem=ag_send_sems.at[s],
                recv_sem=ag_recv_sems.at[s],
                device_id=(my,),
                device_id_type=pl.DeviceIdType.MESH,
            )
            recv.wait_recv()
            part = contribution(dequant_x(ag_ref[s])).astype(jnp.bfloat16)
            m1 = jnp.max(jnp.abs(part), axis=0).astype(jnp.float32)
            mx_arr = jnp.maximum(
                jnp.max(m1, keepdims=True), 1e-20).reshape(1, 1)
            inv = (127.0 / mx_arr).astype(jnp.bfloat16)
            rs_send_ref[s] = jnp.clip(
                jnp.rint(part * inv), -127, 127
            ).astype(jnp.int8).reshape(B, SQ_PER, D)
            sc_send_ref[s] = jnp.full((8, 128), mx_arr[0, 0] / 127.0,
                                      jnp.float32)
            dest = ((my - 1 - s) % N_DEV,)
            for src, dst, ssem, rsem in (
                (rs_send_ref.at[s], rs_recv_ref.at[s],
                 rs_send_sems.at[s], rs_recv_sems.at[s]),
                (sc_send_ref.at[s], sc_recv_ref.at[s],
                 sc_send_sems.at[s], sc_recv_sems.at[s]),
            ):
                rdma = pltpu.make_async_remote_copy(
                    src_ref=src, dst_ref=dst, send_sem=ssem, recv_sem=rsem,
                    device_id=dest, device_id_type=pl.DeviceIdType.MESH,
                )
                rdma.start()
                rs_sends.append(rdma)

        for q_ in range(N_DEV - 1):
            for dst, rsem, src, ssem in (
                (rs_recv_ref.at[q_], rs_recv_sems.at[q_],
                 rs_send_ref.at[q_], rs_send_sems.at[q_]),
                (sc_recv_ref.at[q_], sc_recv_sems.at[q_],
                 sc_send_ref.at[q_], sc_send_sems.at[q_]),
            ):
                recv = pltpu.make_async_remote_copy(
                    src_ref=src, dst_ref=dst, send_sem=ssem, recv_sem=rsem,
                    device_id=(my,), device_id_type=pl.DeviceIdType.MESH,
                )
                recv.wait_recv()
            scale = sc_recv_ref[q_][0, 0]
            acc = acc + rs_recv_ref[q_].reshape(
                B * SQ_PER, D).astype(jnp.float32) * scale
        out_ref[...] = acc.reshape(B, SQ_PER, D)

        for rdma in ag_sends + rs_sends:
            rdma.wait_send()

    return pl.pallas_call(
        body,
        out_shape=jax.ShapeDtypeStruct((B, SQ_PER, D), jnp.float32),
        in_specs=[pl.BlockSpec(memory_space=pltpu.VMEM)] * 5,
        out_specs=pl.BlockSpec(memory_space=pltpu.VMEM),
        scratch_shapes=[
            pltpu.VMEM((B, SQ_PER, D), jnp.int8),
            pltpu.VMEM((N_DEV - 1, B, SQ_PER, D), jnp.int8),
            pltpu.VMEM((N_DEV - 1, B, SQ_PER, D), jnp.int8),
            pltpu.VMEM((N_DEV - 1, B, SQ_PER, D), jnp.int8),
            pltpu.VMEM((N_DEV - 1, 8, 128), jnp.float32),
            pltpu.VMEM((N_DEV - 1, 8, 128), jnp.float32),
            pltpu.SemaphoreType.DMA((N_DEV - 1,)),
            pltpu.SemaphoreType.DMA((N_DEV - 1,)),
            pltpu.SemaphoreType.DMA((N_DEV - 1,)),
            pltpu.SemaphoreType.DMA((N_DEV - 1,)),
            pltpu.SemaphoreType.DMA((N_DEV - 1,)),
            pltpu.SemaphoreType.DMA((N_DEV - 1,)),
        ],
        compiler_params=pltpu.CompilerParams(collective_id=0),
    )(x, Wq, Wo, K_ext, V_ext)
